# Initial kernel scaffold; baseline (speedup 1.0000x reference)
import jax
import jax.numpy as jnp
from jax.experimental import pallas as pl


def kernel(x, edge_index, batch, edge_attr, We0, be0, eps0, m0W1, m0b1, m0g, m0be, m0W2, m0b2, g0, bb0, We1, be1, eps1, m1W1, m1b1, m1g, m1be, m1W2, m1b2, g1, bb1, Wl, bl):
    raise NotImplementedError("write your pallas kernel here")


# speedup vs baseline: 1.6328x; 1.6328x over previous
import functools

import jax
import jax.numpy as jnp
from jax import lax
from jax.experimental import pallas as pl
from jax.experimental.pallas import tpu as pltpu
from jax.experimental.pallas import tpu_sc as plsc

N_NODES = 10000
D = 128
E_TOTAL = 320000
G_GRAPHS = 64
NC = 2
NS = 16
EPT = E_TOTAL // NS
CHUNK = 80
NCHUNK = EPT // CHUNK
HALF = 5056
TRASH = 64
ACC_R = HALF + TRASH
RPT = ACC_R // NS
VPR = D // 16


def _edge_body(x_hbm, src_hbm, dst_hbm, ea_hbm, wb_hbm, out_hbm,
               src_v, dst_v, eab_v, rows_v, wb_v, acc_sh, sem):
    c = lax.axis_index("c")
    s = lax.axis_index("s")

    pltpu.sync_copy(src_hbm.at[s], src_v)
    pltpu.sync_copy(dst_hbm.at[s], dst_v)
    pltpu.sync_copy(wb_hbm, wb_v)

    base = c * HALF

    def _remap(i, carry):
        for j in range(CHUNK // 16):
            dv = dst_v[i, pl.ds(j * 16, 16)]
            local = dv - base
            ok = (local >= 0) & (local < HALF)
            trash = HALF + (dv & (TRASH - 1))
            dst_v[i, pl.ds(j * 16, 16)] = jnp.where(ok, local, trash)
        return carry

    lax.fori_loop(0, NCHUNK, _remap, 0)

    zero = jnp.zeros((16,), jnp.float32)

    def _zrow(i, carry):
        for j in range(VPR):
            rows_v[i, pl.ds(j * 16, 16)] = zero
        return carry

    lax.fori_loop(0, CHUNK, _zrow, 0)

    def _zcopy(i, carry):
        pltpu.sync_copy(rows_v, acc_sh.at[pl.ds(s * RPT + i * CHUNK, CHUNK)])
        return carry

    lax.fori_loop(0, RPT // CHUNK, _zcopy, 0)

    plsc.subcore_barrier()

    w_regs = [wb_v[j] for j in range(VPR)]
    b_regs = [wb_v[VPR + j] for j in range(VPR)]

    def _chunk(ci, carry):
        pltpu.async_copy(x_hbm.at[src_v.at[ci]], rows_v, sem).wait()
        pltpu.sync_copy(ea_hbm.at[s, pl.ds(ci * CHUNK, CHUNK)], eab_v)

        def _sub(si, carry2):
            for e in range(16):
                r = si * 16 + e
                eab = eab_v[r]
                for j in range(VPR):
                    v = rows_v[r, pl.ds(j * 16, 16)]
                    rows_v[r, pl.ds(j * 16, 16)] = jnp.maximum(
                        v + eab * w_regs[j] + b_regs[j], 0.0)
            return carry2

        lax.fori_loop(0, CHUNK // 16, _sub, 0)

        pltpu.sync_copy(rows_v, acc_sh.at[dst_v.at[ci]], add=True)
        return carry

    lax.fori_loop(0, NCHUNK, _chunk, 0)

    plsc.subcore_barrier()

    pltpu.sync_copy(acc_sh.at[pl.ds(s * RPT, RPT)],
                    out_hbm.at[c, pl.ds(s * RPT, RPT)])


@functools.cache
def _make_edge_aggregate():
    return pl.kernel(
        _edge_body,
        out_type=jax.ShapeDtypeStruct((NC, ACC_R, D), jnp.float32),
        mesh=plsc.VectorSubcoreMesh(core_axis_name="c", subcore_axis_name="s",
                                    num_cores=NC, num_subcores=NS),
        scratch_types=[
            pltpu.VMEM((NCHUNK, CHUNK), jnp.int32),
            pltpu.VMEM((NCHUNK, CHUNK), jnp.int32),
            pltpu.VMEM((CHUNK, 16), jnp.float32),
            pltpu.VMEM((CHUNK, D), jnp.float32),
            pltpu.VMEM((2 * VPR, 16), jnp.float32),
            pltpu.VMEM_SHARED((ACC_R, D), jnp.float32),
            pltpu.SemaphoreType.DMA,
        ],
    )


def _edge_aggregate(x, src, dst, ea16, wb):
    return _make_edge_aggregate()(x, src, dst, ea16, wb)


def _agg_from_partials(p_ref):
    return jnp.concatenate(
        [p_ref[0, :HALF], p_ref[1, :N_NODES - HALF]], axis=0)


def _node_body(eps_ref, x_ref, p_ref, W1_ref, b1_ref, g1_ref, be1_ref,
               W2_ref, b2_ref, go_ref, bo_ref, out_ref):
    a = 1.0 + eps_ref[0]
    u = a * x_ref[...] + _agg_from_partials(p_ref)
    h = jnp.dot(u, W1_ref[...], preferred_element_type=jnp.float32) + b1_ref[...]
    m = jnp.mean(h, axis=0, keepdims=True)
    v = jnp.mean((h - m) ** 2, axis=0, keepdims=True)
    h = jnp.maximum(g1_ref[...] * (h - m) * lax.rsqrt(v + 1e-5) + be1_ref[...],
                    0.0)
    h2 = jnp.dot(h, W2_ref[...], preferred_element_type=jnp.float32) + b2_ref[...]
    m2 = jnp.mean(h2, axis=0, keepdims=True)
    v2 = jnp.mean((h2 - m2) ** 2, axis=0, keepdims=True)
    out_ref[...] = jnp.maximum(
        go_ref[...] * (h2 - m2) * lax.rsqrt(v2 + 1e-5) + bo_ref[...], 0.0)


def _node_phase(eps, x, partials, W1, b1, g1, be1, W2, b2, go, bo):
    return pl.pallas_call(
        _node_body,
        out_shape=jax.ShapeDtypeStruct((N_NODES, D), jnp.float32),
        in_specs=[pl.BlockSpec(memory_space=pltpu.SMEM)] +
                 [pl.BlockSpec()] * 10,
    )(eps, x, partials, W1, b1, g1, be1, W2, b2, go, bo)


def _final_body(eps_ref, batch_ref, x_ref, p_ref, W1_ref, b1_ref, g1_ref,
                be1_ref, W2_ref, b2_ref, go_ref, bo_ref, Wl_ref, bl_ref,
                out_ref):
    a = 1.0 + eps_ref[0]
    u = a * x_ref[...] + _agg_from_partials(p_ref)
    h = jnp.dot(u, W1_ref[...], preferred_element_type=jnp.float32) + b1_ref[...]
    m = jnp.mean(h, axis=0, keepdims=True)
    v = jnp.mean((h - m) ** 2, axis=0, keepdims=True)
    h = jnp.maximum(g1_ref[...] * (h - m) * lax.rsqrt(v + 1e-5) + be1_ref[...],
                    0.0)
    h2 = jnp.dot(h, W2_ref[...], preferred_element_type=jnp.float32) + b2_ref[...]
    m2 = jnp.mean(h2, axis=0, keepdims=True)
    v2 = jnp.mean((h2 - m2) ** 2, axis=0, keepdims=True)
    hf = jnp.maximum(
        go_ref[...] * (h2 - m2) * lax.rsqrt(v2 + 1e-5) + bo_ref[...], 0.0)
    onehot = (lax.broadcasted_iota(jnp.int32, (G_GRAPHS, N_NODES), 0)
              == batch_ref[...]).astype(jnp.float32)
    pooled = jnp.dot(onehot, hf, preferred_element_type=jnp.float32)
    out_ref[...] = (jnp.dot(pooled, Wl_ref[...],
                            preferred_element_type=jnp.float32) + bl_ref[...])


def _final_phase(eps, batch, x, partials, W1, b1, g1, be1, W2, b2, go, bo,
                 Wl, bl):
    return pl.pallas_call(
        _final_body,
        out_shape=jax.ShapeDtypeStruct((G_GRAPHS, 10), jnp.float32),
        in_specs=[pl.BlockSpec(memory_space=pltpu.SMEM)] +
                 [pl.BlockSpec()] * 13,
    )(eps, batch, x, partials, W1, b1, g1, be1, W2, b2, go, bo, Wl, bl)


def kernel(x, edge_index, batch, edge_attr,
           We0, be0, eps0, m0W1, m0b1, m0g, m0be, m0W2, m0b2, g0, bb0,
           We1, be1, eps1, m1W1, m1b1, m1g, m1be, m1W2, m1b2, g1, bb1,
           Wl, bl):
    src = edge_index[0].astype(jnp.int32).reshape(NS, NCHUNK, CHUNK)
    dst = edge_index[1].astype(jnp.int32).reshape(NS, NCHUNK, CHUNK)
    ea16 = jnp.broadcast_to(edge_attr[:, None],
                            (E_TOTAL, 16)).reshape(NS, EPT, 16)

    def _wb(We, be):
        return jnp.concatenate([We.reshape(VPR, 16), be.reshape(VPR, 16)], 0)

    p0 = _edge_aggregate(x, src, dst, ea16, _wb(We0, be0))
    h = _node_phase(eps0.reshape(1), x, p0,
                    m0W1, m0b1.reshape(1, D), m0g.reshape(1, D),
                    m0be.reshape(1, D), m0W2, m0b2.reshape(1, D),
                    g0.reshape(1, D), bb0.reshape(1, D))
    p1 = _edge_aggregate(h, src, dst, ea16, _wb(We1, be1))
    out = _final_phase(eps1.reshape(1),
                       batch.astype(jnp.int32).reshape(1, N_NODES),
                       h, p1,
                       m1W1, m1b1.reshape(1, D), m1g.reshape(1, D),
                       m1be.reshape(1, D), m1W2, m1b2.reshape(1, D),
                       g1.reshape(1, D), bb1.reshape(1, D), Wl,
                       bl.reshape(1, 10))
    return out


# speedup vs baseline: 2.5403x; 1.5558x over previous
import functools

import jax
import jax.numpy as jnp
from jax import lax
from jax.experimental import pallas as pl
from jax.experimental.pallas import tpu as pltpu
from jax.experimental.pallas import tpu_sc as plsc

N_NODES = 10000
D = 128
E_TOTAL = 320000
G_GRAPHS = 64
NC = 2
NS = 16
EPT = E_TOTAL // NS
CHUNK = 80
NCHUNK = EPT // CHUNK
HALF = 5056
TRASH = 64
ACC_R = HALF + TRASH
RPT = ACC_R // NS
VPR = D // 16


def _edge_body(x_hbm, src_hbm, dst_hbm, ea_hbm, wb_hbm, out_hbm,
               src_v, dst_v, eab_v, rows_v, wb_v, acc_sh,
               sem0, sem1, esem0, esem1, isem0, isem1):
    c = lax.axis_index("c")
    s = lax.axis_index("s")

    pltpu.sync_copy(wb_hbm, wb_v)

    zero = jnp.zeros((16,), jnp.float32)

    def _zrow(i, carry):
        for j in range(VPR):
            rows_v[0, i, pl.ds(j * 16, 16)] = zero
        return carry

    lax.fori_loop(0, CHUNK, _zrow, 0)

    def _zcopy(i, carry):
        pltpu.sync_copy(rows_v.at[0],
                        acc_sh.at[pl.ds(s * RPT + i * CHUNK, CHUNK)])
        return carry

    lax.fori_loop(0, RPT // CHUNK, _zcopy, 0)

    plsc.subcore_barrier()

    w_regs = [wb_v[j] for j in range(VPR)]
    b_regs = [wb_v[VPR + j] for j in range(VPR)]
    gsems = [sem0, sem1]
    esems = [esem0, esem1]
    isems = [isem0, isem1]
    base = c * HALF

    def _idx_copy(ci, b):
        pltpu.async_copy(src_hbm.at[s, ci], src_v.at[b], isems[b])
        pltpu.async_copy(dst_hbm.at[s, ci], dst_v.at[b], isems[b])

    def _idx_wait(ci, b):
        pltpu.make_async_copy(src_hbm.at[s, ci], src_v.at[b],
                              isems[b]).wait()
        pltpu.make_async_copy(dst_hbm.at[s, ci], dst_v.at[b],
                              isems[b]).wait()

    def _remap(b):
        for j in range(CHUNK // 16):
            dv = dst_v[b, pl.ds(j * 16, 16)]
            local = dv - base
            ok = (local >= 0) & (local < HALF)
            trash = HALF + (dv & (TRASH - 1))
            dst_v[b, pl.ds(j * 16, 16)] = jnp.where(ok, local, trash)

    def _gather(ci, b):
        pltpu.async_copy(x_hbm.at[src_v.at[b]], rows_v.at[b], gsems[b])
        pltpu.async_copy(ea_hbm.at[s, pl.ds(ci * CHUNK, CHUNK)],
                         eab_v.at[b], esems[b])

    def _gwait(ci, b):
        pltpu.make_async_copy(x_hbm.at[src_v.at[b]], rows_v.at[b],
                              gsems[b]).wait()
        pltpu.make_async_copy(ea_hbm.at[s, pl.ds(ci * CHUNK, CHUNK)],
                              eab_v.at[b], esems[b]).wait()

    _idx_copy(0, 0)
    _idx_copy(1, 1)
    _idx_wait(0, 0)
    _remap(0)
    _gather(0, 0)

    def _pair(p, carry):
        for b in range(2):
            ci = 2 * p + b
            _gwait(ci, b)
            nxt = ci + 1

            @pl.when(nxt < NCHUNK)
            def _prefetch():
                _idx_wait(nxt, 1 - b)
                _remap(1 - b)
                _gather(nxt, 1 - b)

            def _sub(si, carry2):
                for e in range(16):
                    r = si * 16 + e
                    eab = eab_v[b, r]
                    for j in range(VPR):
                        v = rows_v[b, r, pl.ds(j * 16, 16)]
                        rows_v[b, r, pl.ds(j * 16, 16)] = jnp.maximum(
                            v + eab * w_regs[j] + b_regs[j], 0.0)
                return carry2

            lax.fori_loop(0, CHUNK // 16, _sub, 0)

            pltpu.sync_copy(rows_v.at[b], acc_sh.at[dst_v.at[b]], add=True)

            @pl.when(ci + 2 < NCHUNK)
            def _nextidx():
                _idx_copy(ci + 2, b)
        return carry

    lax.fori_loop(0, NCHUNK // 2, _pair, 0)

    plsc.subcore_barrier()

    pltpu.sync_copy(acc_sh.at[pl.ds(s * RPT, RPT)],
                    out_hbm.at[c, pl.ds(s * RPT, RPT)])


@functools.cache
def _make_edge_aggregate():
    return pl.kernel(
        _edge_body,
        out_type=jax.ShapeDtypeStruct((NC, ACC_R, D), jnp.float32),
        mesh=plsc.VectorSubcoreMesh(core_axis_name="c", subcore_axis_name="s",
                                    num_cores=NC, num_subcores=NS),
        scratch_types=[
            pltpu.VMEM((2, CHUNK), jnp.int32),
            pltpu.VMEM((2, CHUNK), jnp.int32),
            pltpu.VMEM((2, CHUNK, 16), jnp.float32),
            pltpu.VMEM((2, CHUNK, D), jnp.float32),
            pltpu.VMEM((2 * VPR, 16), jnp.float32),
            pltpu.VMEM_SHARED((ACC_R, D), jnp.float32),
            pltpu.SemaphoreType.DMA,
            pltpu.SemaphoreType.DMA,
            pltpu.SemaphoreType.DMA,
            pltpu.SemaphoreType.DMA,
            pltpu.SemaphoreType.DMA,
            pltpu.SemaphoreType.DMA,
        ],
    )


def _edge_aggregate(x, src, dst, ea16, wb):
    return _make_edge_aggregate()(x, src, dst, ea16, wb)


def _agg_from_partials(p_ref):
    return jnp.concatenate(
        [p_ref[0, :HALF], p_ref[1, :N_NODES - HALF]], axis=0)


def _node_body(eps_ref, x_ref, p_ref, W1_ref, b1_ref, g1_ref, be1_ref,
               W2_ref, b2_ref, go_ref, bo_ref, out_ref):
    a = 1.0 + eps_ref[0]
    u = a * x_ref[...] + _agg_from_partials(p_ref)
    h = jnp.dot(u, W1_ref[...], preferred_element_type=jnp.float32) + b1_ref[...]
    m = jnp.mean(h, axis=0, keepdims=True)
    v = jnp.mean((h - m) ** 2, axis=0, keepdims=True)
    h = jnp.maximum(g1_ref[...] * (h - m) * lax.rsqrt(v + 1e-5) + be1_ref[...],
                    0.0)
    h2 = jnp.dot(h, W2_ref[...], preferred_element_type=jnp.float32) + b2_ref[...]
    m2 = jnp.mean(h2, axis=0, keepdims=True)
    v2 = jnp.mean((h2 - m2) ** 2, axis=0, keepdims=True)
    out_ref[...] = jnp.maximum(
        go_ref[...] * (h2 - m2) * lax.rsqrt(v2 + 1e-5) + bo_ref[...], 0.0)


def _node_phase(eps, x, partials, W1, b1, g1, be1, W2, b2, go, bo):
    return pl.pallas_call(
        _node_body,
        out_shape=jax.ShapeDtypeStruct((N_NODES, D), jnp.float32),
        in_specs=[pl.BlockSpec(memory_space=pltpu.SMEM)] +
                 [pl.BlockSpec()] * 10,
    )(eps, x, partials, W1, b1, g1, be1, W2, b2, go, bo)


def _final_body(eps_ref, batch_ref, x_ref, p_ref, W1_ref, b1_ref, g1_ref,
                be1_ref, W2_ref, b2_ref, go_ref, bo_ref, Wl_ref, bl_ref,
                out_ref):
    a = 1.0 + eps_ref[0]
    u = a * x_ref[...] + _agg_from_partials(p_ref)
    h = jnp.dot(u, W1_ref[...], preferred_element_type=jnp.float32) + b1_ref[...]
    m = jnp.mean(h, axis=0, keepdims=True)
    v = jnp.mean((h - m) ** 2, axis=0, keepdims=True)
    h = jnp.maximum(g1_ref[...] * (h - m) * lax.rsqrt(v + 1e-5) + be1_ref[...],
                    0.0)
    h2 = jnp.dot(h, W2_ref[...], preferred_element_type=jnp.float32) + b2_ref[...]
    m2 = jnp.mean(h2, axis=0, keepdims=True)
    v2 = jnp.mean((h2 - m2) ** 2, axis=0, keepdims=True)
    hf = jnp.maximum(
        go_ref[...] * (h2 - m2) * lax.rsqrt(v2 + 1e-5) + bo_ref[...], 0.0)
    onehot = (lax.broadcasted_iota(jnp.int32, (G_GRAPHS, N_NODES), 0)
              == batch_ref[...]).astype(jnp.float32)
    pooled = jnp.dot(onehot, hf, preferred_element_type=jnp.float32)
    out_ref[...] = (jnp.dot(pooled, Wl_ref[...],
                            preferred_element_type=jnp.float32) + bl_ref[...])


def _final_phase(eps, batch, x, partials, W1, b1, g1, be1, W2, b2, go, bo,
                 Wl, bl):
    return pl.pallas_call(
        _final_body,
        out_shape=jax.ShapeDtypeStruct((G_GRAPHS, 10), jnp.float32),
        in_specs=[pl.BlockSpec(memory_space=pltpu.SMEM)] +
                 [pl.BlockSpec()] * 13,
    )(eps, batch, x, partials, W1, b1, g1, be1, W2, b2, go, bo, Wl, bl)


def kernel(x, edge_index, batch, edge_attr,
           We0, be0, eps0, m0W1, m0b1, m0g, m0be, m0W2, m0b2, g0, bb0,
           We1, be1, eps1, m1W1, m1b1, m1g, m1be, m1W2, m1b2, g1, bb1,
           Wl, bl):
    src = edge_index[0].astype(jnp.int32).reshape(NS, NCHUNK, CHUNK)
    dst = edge_index[1].astype(jnp.int32).reshape(NS, NCHUNK, CHUNK)
    ea16 = jnp.broadcast_to(edge_attr[:, None],
                            (E_TOTAL, 16)).reshape(NS, EPT, 16)

    def _wb(We, be):
        return jnp.concatenate([We.reshape(VPR, 16), be.reshape(VPR, 16)], 0)

    p0 = _edge_aggregate(x, src, dst, ea16, _wb(We0, be0))
    h = _node_phase(eps0.reshape(1), x, p0,
                    m0W1, m0b1.reshape(1, D), m0g.reshape(1, D),
                    m0be.reshape(1, D), m0W2, m0b2.reshape(1, D),
                    g0.reshape(1, D), bb0.reshape(1, D))
    p1 = _edge_aggregate(h, src, dst, ea16, _wb(We1, be1))
    out = _final_phase(eps1.reshape(1),
                       batch.astype(jnp.int32).reshape(1, N_NODES),
                       h, p1,
                       m1W1, m1b1.reshape(1, D), m1g.reshape(1, D),
                       m1be.reshape(1, D), m1W2, m1b2.reshape(1, D),
                       g1.reshape(1, D), bb1.reshape(1, D), Wl,
                       bl.reshape(1, 10))
    return out


# speedup vs baseline: 5.5230x; 2.1741x over previous
import functools

import jax
import jax.numpy as jnp
from jax import lax
from jax.experimental import pallas as pl
from jax.experimental.pallas import tpu as pltpu
from jax.experimental.pallas import tpu_sc as plsc

N_NODES = 10000
D = 128
E_TOTAL = 320000
G_GRAPHS = 64
NC = 2
NS = 16
EPT = E_TOTAL // NS
CHUNK = 80
NCHUNK = EPT // CHUNK
HALF = 5056
TRASH = 64
ACC_R = HALF + TRASH
RPT = ACC_R // NS
VPR = D // 16
NCH_CAP = NCHUNK + 2
CAPB = NCH_CAP * CHUNK

def _bcast_lane(v, e):
    return lax.gather(
        v, jnp.full((16, 1), e, jnp.int32),
        dimension_numbers=lax.GatherDimensionNumbers(
            offset_dims=(), collapsed_slice_dims=(0,), start_index_map=(0,)),
        slice_sizes=(1,),
        mode=lax.GatherScatterMode.PROMISE_IN_BOUNDS)


def _prefix16(x):
    lane = lax.broadcasted_iota(jnp.int32, (16,), 0)
    for k in (1, 2, 4, 8):
        idx = jnp.maximum(lane - k, 0)
        shifted = lax.gather(
            x, idx[:, None],
            dimension_numbers=lax.GatherDimensionNumbers(
                offset_dims=(), collapsed_slice_dims=(0,),
                start_index_map=(0,)),
            slice_sizes=(1,),
            mode=lax.GatherScatterMode.PROMISE_IN_BOUNDS)
        x = x + jnp.where(lane >= k, shifted, 0)
    return x


def _partition_body(src_hbm, dst_hbm, ea_hbm,
                    csrc_hbm, cdst_hbm, cea_hbm, cnt_hbm,
                    sin_v, din_v, ein_v, csrc_v, cdst_v, cea_v, cnt_v,
                    isem0, isem1):
    c = lax.axis_index("c")
    s = lax.axis_index("s")
    base = c * HALF
    isems = [isem0, isem1]

    def _in_copy(ci, b):
        pltpu.async_copy(src_hbm.at[s, ci], sin_v.at[b], isems[b])
        pltpu.async_copy(dst_hbm.at[s, ci], din_v.at[b], isems[b])
        pltpu.async_copy(ea_hbm.at[s, ci], ein_v.at[b], isems[b])

    def _in_wait(ci, b):
        pltpu.make_async_copy(src_hbm.at[s, ci], sin_v.at[b],
                              isems[b]).wait()
        pltpu.make_async_copy(dst_hbm.at[s, ci], din_v.at[b],
                              isems[b]).wait()
        pltpu.make_async_copy(ea_hbm.at[s, ci], ein_v.at[b],
                              isems[b]).wait()

    _in_copy(0, 0)
    _in_copy(1, 1)

    def _pair(p, cur):
        for b in range(2):
            ci = 2 * p + b
            _in_wait(ci, b)

            @pl.when(ci + 2 < NCHUNK)
            def _next():
                _in_copy(ci + 2, b)

            for g in range(CHUNK // 16):
                sv = sin_v[b, pl.ds(g * 16, 16)]
                dv = din_v[b, pl.ds(g * 16, 16)]
                ev = ein_v[b, pl.ds(g * 16, 16)]
                local = dv - base
                ok = (local >= 0) & (local < HALF)
                pos = _prefix16(jnp.where(ok, 1, 0))
                idx = cur + pos - 1
                plsc.store_scatter(csrc_v, [idx], sv, mask=ok)
                plsc.store_scatter(cdst_v, [idx], local, mask=ok)
                plsc.store_scatter(cea_v, [idx], ev, mask=ok)
                cur = cur + pos[15]
        return cur

    cur = lax.fori_loop(0, NCHUNK // 2, _pair, jnp.int32(0))

    cur16 = (cur // 16) * 16
    lane = lax.broadcasted_iota(jnp.int32, (16,), 0)
    for k in range(6):
        pos = cur16 + 16 * k
        keep = (pos + lane) < cur
        csrc_v[pl.ds(pos, 16)] = jnp.where(keep, csrc_v[pl.ds(pos, 16)], 0)
        cdst_v[pl.ds(pos, 16)] = jnp.where(keep, cdst_v[pl.ds(pos, 16)],
                                           HALF)
        cea_v[pl.ds(pos, 16)] = jnp.where(keep, cea_v[pl.ds(pos, 16)], 0.0)

    nch = (cur + CHUNK - 1) // CHUNK
    for k8 in range(8):
        cnt_v[pl.ds(k8 * 16, 16)] = jnp.full((16,), 1, jnp.int32) * nch
    pltpu.sync_copy(cnt_v, cnt_hbm.at[c, s])
    pltpu.sync_copy(csrc_v, csrc_hbm.at[c, s])
    pltpu.sync_copy(cdst_v, cdst_hbm.at[c, s])
    pltpu.sync_copy(cea_v, cea_hbm.at[c, s])


@functools.cache
def _make_partition():
    return pl.kernel(
        _partition_body,
        out_type=(
            jax.ShapeDtypeStruct((NC, NS, CAPB), jnp.int32),
            jax.ShapeDtypeStruct((NC, NS, CAPB), jnp.int32),
            jax.ShapeDtypeStruct((NC, NS, CAPB), jnp.float32),
            jax.ShapeDtypeStruct((NC, NS, 128), jnp.int32),
        ),
        mesh=plsc.VectorSubcoreMesh(core_axis_name="c", subcore_axis_name="s",
                                    num_cores=NC, num_subcores=NS),
        compiler_params=pltpu.CompilerParams(needs_layout_passes=False),
        scratch_types=[
            pltpu.VMEM((2, CHUNK), jnp.int32),
            pltpu.VMEM((2, CHUNK), jnp.int32),
            pltpu.VMEM((2, CHUNK), jnp.float32),
            pltpu.VMEM((CAPB,), jnp.int32),
            pltpu.VMEM((CAPB,), jnp.int32),
            pltpu.VMEM((CAPB,), jnp.float32),
            pltpu.VMEM((128,), jnp.int32),
            pltpu.SemaphoreType.DMA,
            pltpu.SemaphoreType.DMA,
        ],
    )


def _edge_body(x_hbm, csrc_hbm, cdst_hbm, cea_hbm, cnt_hbm, wb_hbm, out_hbm,
               src_v, dst_v, ea_v, rows_v, wb_v, cnt_v, acc_sh,
               sem0, sem1, esem0, esem1, isem0, isem1):
    c = lax.axis_index("c")
    s = lax.axis_index("s")
    w = c * NS + s

    pltpu.sync_copy(wb_hbm, wb_v)
    pltpu.sync_copy(cnt_hbm.at[c, s], cnt_v)
    nch = cnt_v[pl.ds(0, 16)][15]

    zero = jnp.zeros((16,), jnp.float32)

    def _zrow(i, carry):
        for j in range(VPR):
            rows_v[0, i, pl.ds(j * 16, 16)] = zero
        return carry

    lax.fori_loop(0, CHUNK, _zrow, 0)

    def _zcopy(i, carry):
        pltpu.sync_copy(rows_v.at[0],
                        acc_sh.at[pl.ds(s * RPT + i * CHUNK, CHUNK)])
        return carry

    lax.fori_loop(0, RPT // CHUNK, _zcopy, 0)

    plsc.subcore_barrier()

    w_regs = [wb_v[j] for j in range(VPR)]
    b_regs = [wb_v[VPR + j] for j in range(VPR)]
    gsems = [sem0, sem1]
    esems = [esem0, esem1]
    isems = [isem0, isem1]

    def _idx_copy(ci, b):
        pltpu.async_copy(csrc_hbm.at[w, ci], src_v.at[b], isems[b])
        pltpu.async_copy(cdst_hbm.at[w, ci], dst_v.at[b], isems[b])

    def _idx_wait(ci, b):
        pltpu.make_async_copy(csrc_hbm.at[w, ci], src_v.at[b],
                              isems[b]).wait()
        pltpu.make_async_copy(cdst_hbm.at[w, ci], dst_v.at[b],
                              isems[b]).wait()

    def _gather(ci, b):
        pltpu.async_copy(x_hbm.at[src_v.at[b]], rows_v.at[b], gsems[b])
        pltpu.async_copy(cea_hbm.at[w, ci], ea_v.at[b], esems[b])

    def _gwait(ci, b):
        pltpu.make_async_copy(x_hbm.at[src_v.at[b]], rows_v.at[b],
                              gsems[b]).wait()
        pltpu.make_async_copy(cea_hbm.at[w, ci], ea_v.at[b],
                              esems[b]).wait()

    @pl.when(nch > 0)
    def _pro0():
        _idx_copy(0, 0)

    @pl.when(nch > 1)
    def _pro1():
        _idx_copy(1, 1)

    @pl.when(nch > 0)
    def _pro2():
        _idx_wait(0, 0)
        _gather(0, 0)

    def _pair(p, carry):
        for b in range(2):
            ci = 2 * p + b

            @pl.when(ci < nch)
            def _body():
                _gwait(ci, b)
                nxt = ci + 1

                @pl.when(nxt < nch)
                def _prefetch():
                    _idx_wait(nxt, 1 - b)
                    _gather(nxt, 1 - b)

                def _sub(si, carry2):
                    ev = ea_v[b, pl.ds(si * 16, 16)]
                    for e in range(16):
                        r = si * 16 + e
                        eab = _bcast_lane(ev, e)
                        for j in range(VPR):
                            v = rows_v[b, r, pl.ds(j * 16, 16)]
                            rows_v[b, r, pl.ds(j * 16, 16)] = jnp.maximum(
                                v + eab * w_regs[j] + b_regs[j], 0.0)
                    return carry2

                lax.fori_loop(0, CHUNK // 16, _sub, 0)

                pltpu.sync_copy(rows_v.at[b], acc_sh.at[dst_v.at[b]],
                                add=True)

                @pl.when(ci + 2 < nch)
                def _nextidx():
                    _idx_copy(ci + 2, b)
        return carry

    lax.fori_loop(0, (jnp.maximum(nch, 1) + 1) // 2, _pair, 0)

    plsc.subcore_barrier()

    pltpu.sync_copy(acc_sh.at[pl.ds(s * RPT, RPT)],
                    out_hbm.at[c, pl.ds(s * RPT, RPT)])


@functools.cache
def _make_edge_aggregate():
    return pl.kernel(
        _edge_body,
        out_type=jax.ShapeDtypeStruct((NC, ACC_R, D), jnp.float32),
        mesh=plsc.VectorSubcoreMesh(core_axis_name="c", subcore_axis_name="s",
                                    num_cores=NC, num_subcores=NS),
        compiler_params=pltpu.CompilerParams(needs_layout_passes=False),
        scratch_types=[
            pltpu.VMEM((2, CHUNK), jnp.int32),
            pltpu.VMEM((2, CHUNK), jnp.int32),
            pltpu.VMEM((2, CHUNK), jnp.float32),
            pltpu.VMEM((2, CHUNK, D), jnp.float32),
            pltpu.VMEM((2 * VPR, 16), jnp.float32),
            pltpu.VMEM((128,), jnp.int32),
            pltpu.VMEM_SHARED((ACC_R, D), jnp.float32),
            pltpu.SemaphoreType.DMA,
            pltpu.SemaphoreType.DMA,
            pltpu.SemaphoreType.DMA,
            pltpu.SemaphoreType.DMA,
            pltpu.SemaphoreType.DMA,
            pltpu.SemaphoreType.DMA,
        ],
    )


def _edge_aggregate(x, csrc, cdst, cea, cnt, wb):
    return _make_edge_aggregate()(x, csrc, cdst, cea, cnt, wb)


def _agg_from_partials(p_ref):
    return jnp.concatenate(
        [p_ref[0, :HALF], p_ref[1, :N_NODES - HALF]], axis=0)


def _node_body(eps_ref, x_ref, p_ref, W1_ref, b1_ref, g1_ref, be1_ref,
               W2_ref, b2_ref, go_ref, bo_ref, out_ref):
    a = 1.0 + eps_ref[0]
    u = a * x_ref[...] + _agg_from_partials(p_ref)
    h = jnp.dot(u, W1_ref[...], preferred_element_type=jnp.float32) + b1_ref[...]
    m = jnp.mean(h, axis=0, keepdims=True)
    v = jnp.mean((h - m) ** 2, axis=0, keepdims=True)
    h = jnp.maximum(g1_ref[...] * (h - m) * lax.rsqrt(v + 1e-5) + be1_ref[...],
                    0.0)
    h2 = jnp.dot(h, W2_ref[...], preferred_element_type=jnp.float32) + b2_ref[...]
    m2 = jnp.mean(h2, axis=0, keepdims=True)
    v2 = jnp.mean((h2 - m2) ** 2, axis=0, keepdims=True)
    out_ref[...] = jnp.maximum(
        go_ref[...] * (h2 - m2) * lax.rsqrt(v2 + 1e-5) + bo_ref[...], 0.0)


def _node_phase(eps, x, partials, W1, b1, g1, be1, W2, b2, go, bo):
    return pl.pallas_call(
        _node_body,
        out_shape=jax.ShapeDtypeStruct((N_NODES, D), jnp.float32),
        in_specs=[pl.BlockSpec(memory_space=pltpu.SMEM)] +
                 [pl.BlockSpec()] * 10,
    )(eps, x, partials, W1, b1, g1, be1, W2, b2, go, bo)


def _final_body(eps_ref, batch_ref, x_ref, p_ref, W1_ref, b1_ref, g1_ref,
                be1_ref, W2_ref, b2_ref, go_ref, bo_ref, Wl_ref, bl_ref,
                out_ref):
    a = 1.0 + eps_ref[0]
    u = a * x_ref[...] + _agg_from_partials(p_ref)
    h = jnp.dot(u, W1_ref[...], preferred_element_type=jnp.float32) + b1_ref[...]
    m = jnp.mean(h, axis=0, keepdims=True)
    v = jnp.mean((h - m) ** 2, axis=0, keepdims=True)
    h = jnp.maximum(g1_ref[...] * (h - m) * lax.rsqrt(v + 1e-5) + be1_ref[...],
                    0.0)
    h2 = jnp.dot(h, W2_ref[...], preferred_element_type=jnp.float32) + b2_ref[...]
    m2 = jnp.mean(h2, axis=0, keepdims=True)
    v2 = jnp.mean((h2 - m2) ** 2, axis=0, keepdims=True)
    hf = jnp.maximum(
        go_ref[...] * (h2 - m2) * lax.rsqrt(v2 + 1e-5) + bo_ref[...], 0.0)
    onehot = (lax.broadcasted_iota(jnp.int32, (G_GRAPHS, N_NODES), 0)
              == batch_ref[...]).astype(jnp.float32)
    pooled = jnp.dot(onehot, hf, preferred_element_type=jnp.float32)
    out_ref[...] = (jnp.dot(pooled, Wl_ref[...],
                            preferred_element_type=jnp.float32) + bl_ref[...])


def _final_phase(eps, batch, x, partials, W1, b1, g1, be1, W2, b2, go, bo,
                 Wl, bl):
    return pl.pallas_call(
        _final_body,
        out_shape=jax.ShapeDtypeStruct((G_GRAPHS, 10), jnp.float32),
        in_specs=[pl.BlockSpec(memory_space=pltpu.SMEM)] +
                 [pl.BlockSpec()] * 13,
    )(eps, batch, x, partials, W1, b1, g1, be1, W2, b2, go, bo, Wl, bl)


def kernel(x, edge_index, batch, edge_attr,
           We0, be0, eps0, m0W1, m0b1, m0g, m0be, m0W2, m0b2, g0, bb0,
           We1, be1, eps1, m1W1, m1b1, m1g, m1be, m1W2, m1b2, g1, bb1,
           Wl, bl):
    src = edge_index[0].astype(jnp.int32).reshape(NS, NCHUNK, CHUNK)
    dst = edge_index[1].astype(jnp.int32).reshape(NS, NCHUNK, CHUNK)
    ea = edge_attr.reshape(NS, NCHUNK, CHUNK)

    csrc, cdst, cea, cnt = _make_partition()(src, dst, ea)
    csrc, cdst, cea = lax.optimization_barrier((csrc, cdst, cea))
    csrc = csrc.reshape(NC * NS, NCH_CAP, CHUNK)
    cdst = cdst.reshape(NC * NS, NCH_CAP, CHUNK)
    cea = cea.reshape(NC * NS, NCH_CAP, CHUNK)

    def _wb(We, be):
        return jnp.concatenate([We.reshape(VPR, 16), be.reshape(VPR, 16)], 0)

    p0 = _edge_aggregate(x, csrc, cdst, cea, cnt, _wb(We0, be0))
    h = _node_phase(eps0.reshape(1), x, p0,
                    m0W1, m0b1.reshape(1, D), m0g.reshape(1, D),
                    m0be.reshape(1, D), m0W2, m0b2.reshape(1, D),
                    g0.reshape(1, D), bb0.reshape(1, D))
    p1 = _edge_aggregate(h, csrc, cdst, cea, cnt, _wb(We1, be1))
    out = _final_phase(eps1.reshape(1),
                       batch.astype(jnp.int32).reshape(1, N_NODES),
                       h, p1,
                       m1W1, m1b1.reshape(1, D), m1g.reshape(1, D),
                       m1be.reshape(1, D), m1W2, m1b2.reshape(1, D),
                       g1.reshape(1, D), bb1.reshape(1, D), Wl,
                       bl.reshape(1, 10))
    return out


# speedup vs baseline: 6.1313x; 1.1101x over previous
import functools

import jax
import jax.numpy as jnp
from jax import lax
from jax.experimental import pallas as pl
from jax.experimental.pallas import tpu as pltpu
from jax.experimental.pallas import tpu_sc as plsc

N_NODES = 10000
D = 128
E_TOTAL = 320000
G_GRAPHS = 64
NC = 2
NS = 16
EPT = E_TOTAL // NS
CHUNK = 80
NCHUNK = EPT // CHUNK
HALF = 5056
TRASH = 64
ACC_R = HALF + TRASH
RPT = ACC_R // NS
VPR = D // 16
CCH = 128
NCH_CAP = EPT // CCH + 2
CAPB = NCH_CAP * CCH
DEPTH = 4

def _bcast_lane(v, e):
    return lax.gather(
        v, jnp.full((16, 1), e, jnp.int32),
        dimension_numbers=lax.GatherDimensionNumbers(
            offset_dims=(), collapsed_slice_dims=(0,), start_index_map=(0,)),
        slice_sizes=(1,),
        mode=lax.GatherScatterMode.PROMISE_IN_BOUNDS)


def _prefix16(x):
    lane = lax.broadcasted_iota(jnp.int32, (16,), 0)
    for k in (1, 2, 4, 8):
        idx = jnp.maximum(lane - k, 0)
        shifted = lax.gather(
            x, idx[:, None],
            dimension_numbers=lax.GatherDimensionNumbers(
                offset_dims=(), collapsed_slice_dims=(0,),
                start_index_map=(0,)),
            slice_sizes=(1,),
            mode=lax.GatherScatterMode.PROMISE_IN_BOUNDS)
        x = x + jnp.where(lane >= k, shifted, 0)
    return x


def _partition_body(src_hbm, dst_hbm, ea_hbm,
                    csrc_hbm, cdst_hbm, cea_hbm, cnt_hbm,
                    sin_v, din_v, ein_v, csrc_v, cdst_v, cea_v, cnt_v,
                    isem0, isem1):
    c = lax.axis_index("c")
    s = lax.axis_index("s")
    base = c * HALF
    isems = [isem0, isem1]

    def _in_copy(ci, b):
        pltpu.async_copy(src_hbm.at[s, ci], sin_v.at[b], isems[b])
        pltpu.async_copy(dst_hbm.at[s, ci], din_v.at[b], isems[b])
        pltpu.async_copy(ea_hbm.at[s, ci], ein_v.at[b], isems[b])

    def _in_wait(ci, b):
        pltpu.make_async_copy(src_hbm.at[s, ci], sin_v.at[b],
                              isems[b]).wait()
        pltpu.make_async_copy(dst_hbm.at[s, ci], din_v.at[b],
                              isems[b]).wait()
        pltpu.make_async_copy(ea_hbm.at[s, ci], ein_v.at[b],
                              isems[b]).wait()

    _in_copy(0, 0)
    _in_copy(1, 1)

    def _pair(p, cur):
        for b in range(2):
            ci = 2 * p + b
            _in_wait(ci, b)

            @pl.when(ci + 2 < NCHUNK)
            def _next():
                _in_copy(ci + 2, b)

            for g in range(CHUNK // 16):
                sv = sin_v[b, pl.ds(g * 16, 16)]
                dv = din_v[b, pl.ds(g * 16, 16)]
                ev = ein_v[b, pl.ds(g * 16, 16)]
                local = dv - base
                ok = (local >= 0) & (local < HALF)
                pos = _prefix16(jnp.where(ok, 1, 0))
                idx = cur + pos - 1
                plsc.store_scatter(csrc_v, [idx], sv, mask=ok)
                plsc.store_scatter(cdst_v, [idx], local, mask=ok)
                plsc.store_scatter(cea_v, [idx], ev, mask=ok)
                cur = cur + pos[15]
        return cur

    cur = lax.fori_loop(0, NCHUNK // 2, _pair, jnp.int32(0))

    cur16 = (cur // 16) * 16
    lane = lax.broadcasted_iota(jnp.int32, (16,), 0)
    for k in range(9):
        pos = cur16 + 16 * k
        keep = (pos + lane) < cur
        csrc_v[pl.ds(pos, 16)] = jnp.where(keep, csrc_v[pl.ds(pos, 16)], 0)
        cdst_v[pl.ds(pos, 16)] = jnp.where(keep, cdst_v[pl.ds(pos, 16)],
                                           HALF)
        cea_v[pl.ds(pos, 16)] = jnp.where(keep, cea_v[pl.ds(pos, 16)], 0.0)

    nch = (cur + CCH - 1) // CCH
    for k8 in range(8):
        cnt_v[pl.ds(k8 * 16, 16)] = jnp.full((16,), 1, jnp.int32) * nch
    pltpu.sync_copy(cnt_v, cnt_hbm.at[c, s])
    pltpu.sync_copy(csrc_v, csrc_hbm.at[c, s])
    pltpu.sync_copy(cdst_v, cdst_hbm.at[c, s])
    pltpu.sync_copy(cea_v, cea_hbm.at[c, s])


@functools.cache
def _make_partition():
    return pl.kernel(
        _partition_body,
        out_type=(
            jax.ShapeDtypeStruct((NC, NS, CAPB), jnp.int32),
            jax.ShapeDtypeStruct((NC, NS, CAPB), jnp.int32),
            jax.ShapeDtypeStruct((NC, NS, CAPB), jnp.float32),
            jax.ShapeDtypeStruct((NC, NS, 128), jnp.int32),
        ),
        mesh=plsc.VectorSubcoreMesh(core_axis_name="c", subcore_axis_name="s",
                                    num_cores=NC, num_subcores=NS),
        compiler_params=pltpu.CompilerParams(needs_layout_passes=False),
        scratch_types=[
            pltpu.VMEM((2, CHUNK), jnp.int32),
            pltpu.VMEM((2, CHUNK), jnp.int32),
            pltpu.VMEM((2, CHUNK), jnp.float32),
            pltpu.VMEM((CAPB,), jnp.int32),
            pltpu.VMEM((CAPB,), jnp.int32),
            pltpu.VMEM((CAPB,), jnp.float32),
            pltpu.VMEM((128,), jnp.int32),
            pltpu.SemaphoreType.DMA,
            pltpu.SemaphoreType.DMA,
        ],
    )


def _edge_body(x_hbm, csrc_hbm, cdst_hbm, cea_hbm, cnt_hbm, wb_hbm, out_hbm,
               src_v, dst_v, ea_v, rows_v, wb_v, cnt_v, acc_sh, *sems):
    c = lax.axis_index("c")
    s = lax.axis_index("s")
    w = c * NS + s
    isems = sems[0:DEPTH]
    gsems = sems[DEPTH:2 * DEPTH]
    ssems = sems[2 * DEPTH:3 * DEPTH]

    pltpu.sync_copy(wb_hbm, wb_v)
    pltpu.sync_copy(cnt_hbm.at[c, s], cnt_v)
    nch = cnt_v[pl.ds(0, 16)][15]

    zero = jnp.zeros((16,), jnp.float32)

    def _zrow(i, carry):
        for j in range(VPR):
            rows_v[0, i, pl.ds(j * 16, 16)] = zero
        return carry

    lax.fori_loop(0, CCH, _zrow, 0)

    def _zcopy(i, carry):
        pltpu.sync_copy(rows_v.at[0],
                        acc_sh.at[pl.ds(s * RPT + i * CCH, CCH)])
        return carry

    lax.fori_loop(0, RPT // CCH, _zcopy, 0)
    pltpu.sync_copy(rows_v.at[0, pl.ds(0, RPT % CCH)],
                    acc_sh.at[pl.ds(s * RPT + (RPT // CCH) * CCH,
                                    RPT % CCH)])

    plsc.subcore_barrier()

    w_regs = [wb_v[j] for j in range(VPR)]
    b_regs = [wb_v[VPR + j] for j in range(VPR)]

    def _idx_copy(ci, b):
        pltpu.async_copy(csrc_hbm.at[w, ci], src_v.at[b], isems[b])
        pltpu.async_copy(cdst_hbm.at[w, ci], dst_v.at[b], isems[b])
        pltpu.async_copy(cea_hbm.at[w, ci], ea_v.at[b], isems[b])

    def _idx_wait(ci, b):
        pltpu.make_async_copy(csrc_hbm.at[w, ci], src_v.at[b],
                              isems[b]).wait()
        pltpu.make_async_copy(cdst_hbm.at[w, ci], dst_v.at[b],
                              isems[b]).wait()
        pltpu.make_async_copy(cea_hbm.at[w, ci], ea_v.at[b],
                              isems[b]).wait()

    def _gather(b):
        pltpu.async_copy(x_hbm.at[src_v.at[b]], rows_v.at[b], gsems[b])

    def _gwait(b):
        pltpu.make_async_copy(x_hbm.at[src_v.at[b]], rows_v.at[b],
                              gsems[b]).wait()

    def _scatter(b):
        pltpu.async_copy(rows_v.at[b], acc_sh.at[dst_v.at[b]], ssems[b],
                         add=True)

    def _swait(b):
        pltpu.make_async_copy(rows_v.at[b], acc_sh.at[dst_v.at[b]],
                              ssems[b]).wait()

    for k in range(3):
        @pl.when(nch > k)
        def _pro(k=k):
            _idx_copy(k, k)

    @pl.when(nch > 0)
    def _pro2():
        _idx_wait(0, 0)
        _gather(0)

    def _quad(q, carry):
        for b in range(DEPTH):
            ci = 4 * q + b

            @pl.when(ci < nch)
            def _body(ci=ci, b=b):
                _gwait(b)
                nxt = ci + 1
                b1 = (b + 1) % DEPTH

                @pl.when(nxt < nch)
                def _prefetch():
                    _idx_wait(nxt, b1)
                    _gather(b1)

                def _sub(si, carry2):
                    ev = ea_v[b, pl.ds(si * 16, 16)]
                    for e in range(16):
                        r = si * 16 + e
                        eab = _bcast_lane(ev, e)
                        for j in range(VPR):
                            v = rows_v[b, r, pl.ds(j * 16, 16)]
                            rows_v[b, r, pl.ds(j * 16, 16)] = jnp.maximum(
                                v + eab * w_regs[j] + b_regs[j], 0.0)
                    return carry2

                lax.fori_loop(0, CCH // 16, _sub, 0)

                _scatter(b)

                b3 = (b + 3) % DEPTH

                @pl.when(ci + 3 < nch)
                def _nextidx():
                    @pl.when(ci >= 1)
                    def _drain_prev():
                        _swait(b3)
                    _idx_copy(ci + 3, b3)
        return carry

    lax.fori_loop(0, (jnp.maximum(nch, 1) + DEPTH - 1) // DEPTH, _quad, 0)

    for b in range(DEPTH):
        @pl.when(b < nch)
        def _drain(b=b):
            _swait(b)

    plsc.subcore_barrier()

    pltpu.sync_copy(acc_sh.at[pl.ds(s * RPT, RPT)],
                    out_hbm.at[c, pl.ds(s * RPT, RPT)])


@functools.cache
def _make_edge_aggregate():
    return pl.kernel(
        _edge_body,
        out_type=jax.ShapeDtypeStruct((NC, ACC_R, D), jnp.float32),
        mesh=plsc.VectorSubcoreMesh(core_axis_name="c", subcore_axis_name="s",
                                    num_cores=NC, num_subcores=NS),
        compiler_params=pltpu.CompilerParams(needs_layout_passes=False),
        scratch_types=[
            pltpu.VMEM((DEPTH, CCH), jnp.int32),
            pltpu.VMEM((DEPTH, CCH), jnp.int32),
            pltpu.VMEM((DEPTH, CCH), jnp.float32),
            pltpu.VMEM((DEPTH, CCH, D), jnp.float32),
            pltpu.VMEM((2 * VPR, 16), jnp.float32),
            pltpu.VMEM((128,), jnp.int32),
            pltpu.VMEM_SHARED((ACC_R, D), jnp.float32),
        ] + [pltpu.SemaphoreType.DMA] * (3 * DEPTH),
    )


def _edge_aggregate(x, csrc, cdst, cea, cnt, wb):
    return _make_edge_aggregate()(x, csrc, cdst, cea, cnt, wb)


def _agg_from_partials(p_ref):
    return jnp.concatenate(
        [p_ref[0, :HALF], p_ref[1, :N_NODES - HALF]], axis=0)


def _node_body(eps_ref, x_ref, p_ref, W1_ref, b1_ref, g1_ref, be1_ref,
               W2_ref, b2_ref, go_ref, bo_ref, out_ref):
    a = 1.0 + eps_ref[0]
    u = a * x_ref[...] + _agg_from_partials(p_ref)
    h = jnp.dot(u, W1_ref[...], preferred_element_type=jnp.float32) + b1_ref[...]
    m = jnp.mean(h, axis=0, keepdims=True)
    v = jnp.mean((h - m) ** 2, axis=0, keepdims=True)
    h = jnp.maximum(g1_ref[...] * (h - m) * lax.rsqrt(v + 1e-5) + be1_ref[...],
                    0.0)
    h2 = jnp.dot(h, W2_ref[...], preferred_element_type=jnp.float32) + b2_ref[...]
    m2 = jnp.mean(h2, axis=0, keepdims=True)
    v2 = jnp.mean((h2 - m2) ** 2, axis=0, keepdims=True)
    out_ref[...] = jnp.maximum(
        go_ref[...] * (h2 - m2) * lax.rsqrt(v2 + 1e-5) + bo_ref[...], 0.0)


def _node_phase(eps, x, partials, W1, b1, g1, be1, W2, b2, go, bo):
    return pl.pallas_call(
        _node_body,
        out_shape=jax.ShapeDtypeStruct((N_NODES, D), jnp.float32),
        in_specs=[pl.BlockSpec(memory_space=pltpu.SMEM)] +
                 [pl.BlockSpec()] * 10,
    )(eps, x, partials, W1, b1, g1, be1, W2, b2, go, bo)


def _final_body(eps_ref, batch_ref, x_ref, p_ref, W1_ref, b1_ref, g1_ref,
                be1_ref, W2_ref, b2_ref, go_ref, bo_ref, Wl_ref, bl_ref,
                out_ref):
    a = 1.0 + eps_ref[0]
    u = a * x_ref[...] + _agg_from_partials(p_ref)
    h = jnp.dot(u, W1_ref[...], preferred_element_type=jnp.float32) + b1_ref[...]
    m = jnp.mean(h, axis=0, keepdims=True)
    v = jnp.mean((h - m) ** 2, axis=0, keepdims=True)
    h = jnp.maximum(g1_ref[...] * (h - m) * lax.rsqrt(v + 1e-5) + be1_ref[...],
                    0.0)
    h2 = jnp.dot(h, W2_ref[...], preferred_element_type=jnp.float32) + b2_ref[...]
    m2 = jnp.mean(h2, axis=0, keepdims=True)
    v2 = jnp.mean((h2 - m2) ** 2, axis=0, keepdims=True)
    hf = jnp.maximum(
        go_ref[...] * (h2 - m2) * lax.rsqrt(v2 + 1e-5) + bo_ref[...], 0.0)
    onehot = (lax.broadcasted_iota(jnp.int32, (G_GRAPHS, N_NODES), 0)
              == batch_ref[...]).astype(jnp.float32)
    pooled = jnp.dot(onehot, hf, preferred_element_type=jnp.float32)
    out_ref[...] = (jnp.dot(pooled, Wl_ref[...],
                            preferred_element_type=jnp.float32) + bl_ref[...])


def _final_phase(eps, batch, x, partials, W1, b1, g1, be1, W2, b2, go, bo,
                 Wl, bl):
    return pl.pallas_call(
        _final_body,
        out_shape=jax.ShapeDtypeStruct((G_GRAPHS, 10), jnp.float32),
        in_specs=[pl.BlockSpec(memory_space=pltpu.SMEM)] +
                 [pl.BlockSpec()] * 13,
    )(eps, batch, x, partials, W1, b1, g1, be1, W2, b2, go, bo, Wl, bl)


def kernel(x, edge_index, batch, edge_attr,
           We0, be0, eps0, m0W1, m0b1, m0g, m0be, m0W2, m0b2, g0, bb0,
           We1, be1, eps1, m1W1, m1b1, m1g, m1be, m1W2, m1b2, g1, bb1,
           Wl, bl):
    src = edge_index[0].astype(jnp.int32).reshape(NS, NCHUNK, CHUNK)
    dst = edge_index[1].astype(jnp.int32).reshape(NS, NCHUNK, CHUNK)
    ea = edge_attr.reshape(NS, NCHUNK, CHUNK)

    csrc, cdst, cea, cnt = _make_partition()(src, dst, ea)
    csrc, cdst, cea = lax.optimization_barrier((csrc, cdst, cea))
    csrc = csrc.reshape(NC * NS, NCH_CAP, CCH)
    cdst = cdst.reshape(NC * NS, NCH_CAP, CCH)
    cea = cea.reshape(NC * NS, NCH_CAP, CCH)

    def _wb(We, be):
        return jnp.concatenate([We.reshape(VPR, 16), be.reshape(VPR, 16)], 0)

    p0 = _edge_aggregate(x, csrc, cdst, cea, cnt, _wb(We0, be0))
    h = _node_phase(eps0.reshape(1), x, p0,
                    m0W1, m0b1.reshape(1, D), m0g.reshape(1, D),
                    m0be.reshape(1, D), m0W2, m0b2.reshape(1, D),
                    g0.reshape(1, D), bb0.reshape(1, D))
    p1 = _edge_aggregate(h, csrc, cdst, cea, cnt, _wb(We1, be1))
    out = _final_phase(eps1.reshape(1),
                       batch.astype(jnp.int32).reshape(1, N_NODES),
                       h, p1,
                       m1W1, m1b1.reshape(1, D), m1g.reshape(1, D),
                       m1be.reshape(1, D), m1W2, m1b2.reshape(1, D),
                       g1.reshape(1, D), bb1.reshape(1, D), Wl,
                       bl.reshape(1, 10))
    return out


# speedup vs baseline: 6.1696x; 1.0063x over previous
import functools

import jax
import jax.numpy as jnp
from jax import lax
from jax.experimental import pallas as pl
from jax.experimental.pallas import tpu as pltpu
from jax.experimental.pallas import tpu_sc as plsc

N_NODES = 10000
D = 128
E_TOTAL = 320000
G_GRAPHS = 64
NC = 2
NS = 16
EPT = E_TOTAL // NS
CHUNK = 80
NCHUNK = EPT // CHUNK
HALF = 5056
TRASH = 64
ACC_R = HALF + TRASH
RPT = ACC_R // NS
VPR = D // 16
RAWC = 80
NRAW = EPT // RAWC
CCH = 128
NCH_CAP = EPT // CCH + 2
CAPB = NCH_CAP * CCH
DEPTH = 4

def _bcast_lane(v, e):
    return lax.gather(
        v, jnp.full((16, 1), e, jnp.int32),
        dimension_numbers=lax.GatherDimensionNumbers(
            offset_dims=(), collapsed_slice_dims=(0,), start_index_map=(0,)),
        slice_sizes=(1,),
        mode=lax.GatherScatterMode.PROMISE_IN_BOUNDS)


def _prefix16(x):
    lane = lax.broadcasted_iota(jnp.int32, (16,), 0)
    for k in (1, 2, 4, 8):
        idx = jnp.maximum(lane - k, 0)
        shifted = lax.gather(
            x, idx[:, None],
            dimension_numbers=lax.GatherDimensionNumbers(
                offset_dims=(), collapsed_slice_dims=(0,),
                start_index_map=(0,)),
            slice_sizes=(1,),
            mode=lax.GatherScatterMode.PROMISE_IN_BOUNDS)
        x = x + jnp.where(lane >= k, shifted, 0)
    return x


def _partition_body(src_hbm, dst_hbm, ea_hbm,
                    csrc_hbm, cdst_hbm, cea_hbm, cnt_hbm,
                    sin_v, din_v, ein_v, csrc_v, cdst_v, cea_v, cnt_v,
                    isem0, isem1):
    c = lax.axis_index("c")
    s = lax.axis_index("s")
    base = c * HALF
    isems = [isem0, isem1]

    def _in_copy(ci, b):
        pltpu.async_copy(src_hbm.at[s, ci], sin_v.at[b], isems[b])
        pltpu.async_copy(dst_hbm.at[s, ci], din_v.at[b], isems[b])
        pltpu.async_copy(ea_hbm.at[s, ci], ein_v.at[b], isems[b])

    def _in_wait(ci, b):
        pltpu.make_async_copy(src_hbm.at[s, ci], sin_v.at[b],
                              isems[b]).wait()
        pltpu.make_async_copy(dst_hbm.at[s, ci], din_v.at[b],
                              isems[b]).wait()
        pltpu.make_async_copy(ea_hbm.at[s, ci], ein_v.at[b],
                              isems[b]).wait()

    _in_copy(0, 0)
    _in_copy(1, 1)

    def _pair(p, cur):
        for b in range(2):
            ci = 2 * p + b
            _in_wait(ci, b)

            @pl.when(ci + 2 < NRAW)
            def _next():
                _in_copy(ci + 2, b)

            for g in range(RAWC // 16):
                sv = sin_v[b, pl.ds(g * 16, 16)]
                dv = din_v[b, pl.ds(g * 16, 16)]
                ev = ein_v[b, pl.ds(g * 16, 16)]
                local = dv - base
                ok = (local >= 0) & (local < HALF)
                pos = _prefix16(jnp.where(ok, 1, 0))
                idx = cur + pos - 1
                row = lax.shift_right_logical(idx, 7)
                col = idx & (CCH - 1)
                plsc.store_scatter(csrc_v, [row, col], sv, mask=ok)
                plsc.store_scatter(cdst_v, [row, col], local, mask=ok)
                plsc.store_scatter(cea_v, [row, col], ev, mask=ok)
                cur = cur + pos[15]
        return cur

    cur = lax.fori_loop(0, NRAW // 2, _pair, jnp.int32(0))

    cur16 = (cur // 16) * 16
    lane = lax.broadcasted_iota(jnp.int32, (16,), 0)
    zero_i = jnp.zeros((16,), jnp.int32)
    half_i = jnp.full((16,), HALF, jnp.int32)
    zero_f = jnp.zeros((16,), jnp.float32)
    for k in range(9):
        p16 = cur16 + 16 * k + lane
        pad = p16 >= cur
        row = lax.shift_right_logical(p16, 7)
        col = p16 & (CCH - 1)
        plsc.store_scatter(csrc_v, [row, col], zero_i, mask=pad)
        plsc.store_scatter(cdst_v, [row, col], half_i, mask=pad)
        plsc.store_scatter(cea_v, [row, col], zero_f, mask=pad)

    w = c * NS + s
    nch = (cur + CCH - 1) // CCH
    for k8 in range(8):
        cnt_v[pl.ds(k8 * 16, 16)] = jnp.full((16,), 1, jnp.int32) * nch
    pltpu.sync_copy(cnt_v, cnt_hbm.at[c, s])
    pltpu.sync_copy(csrc_v, csrc_hbm.at[w])
    pltpu.sync_copy(cdst_v, cdst_hbm.at[w])
    pltpu.sync_copy(cea_v, cea_hbm.at[w])


@functools.cache
def _make_partition():
    return pl.kernel(
        _partition_body,
        out_type=(
            jax.ShapeDtypeStruct((NC * NS, NCH_CAP, CCH), jnp.int32),
            jax.ShapeDtypeStruct((NC * NS, NCH_CAP, CCH), jnp.int32),
            jax.ShapeDtypeStruct((NC * NS, NCH_CAP, CCH), jnp.float32),
            jax.ShapeDtypeStruct((NC, NS, 128), jnp.int32),
        ),
        mesh=plsc.VectorSubcoreMesh(core_axis_name="c", subcore_axis_name="s",
                                    num_cores=NC, num_subcores=NS),
        compiler_params=pltpu.CompilerParams(needs_layout_passes=False),
        scratch_types=[
            pltpu.VMEM((2, RAWC), jnp.int32),
            pltpu.VMEM((2, RAWC), jnp.int32),
            pltpu.VMEM((2, RAWC), jnp.float32),
            pltpu.VMEM((NCH_CAP, CCH), jnp.int32),
            pltpu.VMEM((NCH_CAP, CCH), jnp.int32),
            pltpu.VMEM((NCH_CAP, CCH), jnp.float32),
            pltpu.VMEM((128,), jnp.int32),
            pltpu.SemaphoreType.DMA,
            pltpu.SemaphoreType.DMA,
        ],
    )


def _edge_body(x_hbm, csrc_hbm, cdst_hbm, cea_hbm, cnt_hbm, wb_hbm, out_hbm,
               src_v, dst_v, ea_v, rows_v, wb_v, cnt_v, acc_sh, *sems):
    c = lax.axis_index("c")
    s = lax.axis_index("s")
    w = c * NS + s
    isems = sems[0:DEPTH]
    gsems = sems[DEPTH:2 * DEPTH]
    ssems = sems[2 * DEPTH:3 * DEPTH]

    pltpu.sync_copy(wb_hbm, wb_v)
    pltpu.sync_copy(cnt_hbm.at[c, s], cnt_v)
    nch = cnt_v[pl.ds(0, 16)][15]

    zero = jnp.zeros((16,), jnp.float32)

    def _zrow(i, carry):
        for j in range(VPR):
            rows_v[0, i, pl.ds(j * 16, 16)] = zero
        return carry

    lax.fori_loop(0, CCH, _zrow, 0)

    def _zcopy(i, carry):
        pltpu.sync_copy(rows_v.at[0],
                        acc_sh.at[pl.ds(s * RPT + i * CCH, CCH)])
        return carry

    lax.fori_loop(0, RPT // CCH, _zcopy, 0)
    pltpu.sync_copy(rows_v.at[0, pl.ds(0, RPT % CCH)],
                    acc_sh.at[pl.ds(s * RPT + (RPT // CCH) * CCH,
                                    RPT % CCH)])

    plsc.subcore_barrier()

    w_regs = [wb_v[j] for j in range(VPR)]
    b_regs = [wb_v[VPR + j] for j in range(VPR)]

    def _idx_copy(ci, b):
        pltpu.async_copy(csrc_hbm.at[w, ci], src_v.at[b], isems[b])
        pltpu.async_copy(cdst_hbm.at[w, ci], dst_v.at[b], isems[b])
        pltpu.async_copy(cea_hbm.at[w, ci], ea_v.at[b], isems[b])

    def _idx_wait(ci, b):
        pltpu.make_async_copy(csrc_hbm.at[w, ci], src_v.at[b],
                              isems[b]).wait()
        pltpu.make_async_copy(cdst_hbm.at[w, ci], dst_v.at[b],
                              isems[b]).wait()
        pltpu.make_async_copy(cea_hbm.at[w, ci], ea_v.at[b],
                              isems[b]).wait()

    def _gather(b):
        pltpu.async_copy(x_hbm.at[src_v.at[b]], rows_v.at[b], gsems[b])

    def _gwait(b):
        pltpu.make_async_copy(x_hbm.at[src_v.at[b]], rows_v.at[b],
                              gsems[b]).wait()

    def _scatter(b):
        pltpu.async_copy(rows_v.at[b], acc_sh.at[dst_v.at[b]], ssems[b],
                         add=True)

    def _swait(b):
        pltpu.make_async_copy(rows_v.at[b], acc_sh.at[dst_v.at[b]],
                              ssems[b]).wait()

    for k in range(3):
        @pl.when(nch > k)
        def _pro(k=k):
            _idx_copy(k, k)

    @pl.when(nch > 0)
    def _pro2():
        _idx_wait(0, 0)
        _gather(0)

    def _quad(q, carry):
        for b in range(DEPTH):
            ci = 4 * q + b

            @pl.when(ci < nch)
            def _body(ci=ci, b=b):
                _gwait(b)
                nxt = ci + 1
                b1 = (b + 1) % DEPTH

                @pl.when(nxt < nch)
                def _prefetch():
                    _idx_wait(nxt, b1)
                    _gather(b1)

                def _sub(si, carry2):
                    ev = ea_v[b, pl.ds(si * 16, 16)]
                    for e in range(16):
                        r = si * 16 + e
                        eab = _bcast_lane(ev, e)
                        for j in range(VPR):
                            v = rows_v[b, r, pl.ds(j * 16, 16)]
                            rows_v[b, r, pl.ds(j * 16, 16)] = jnp.maximum(
                                v + eab * w_regs[j] + b_regs[j], 0.0)
                    return carry2

                lax.fori_loop(0, CCH // 16, _sub, 0)

                _scatter(b)

                b3 = (b + 3) % DEPTH

                @pl.when(ci + 3 < nch)
                def _nextidx():
                    @pl.when(ci >= 1)
                    def _drain_prev():
                        _swait(b3)
                    _idx_copy(ci + 3, b3)
        return carry

    lax.fori_loop(0, (jnp.maximum(nch, 1) + DEPTH - 1) // DEPTH, _quad, 0)

    for b in range(DEPTH):
        @pl.when(b < nch)
        def _drain(b=b):
            _swait(b)

    plsc.subcore_barrier()

    pltpu.sync_copy(acc_sh.at[pl.ds(s * RPT, RPT)],
                    out_hbm.at[c, pl.ds(s * RPT, RPT)])


@functools.cache
def _make_edge_aggregate():
    return pl.kernel(
        _edge_body,
        out_type=jax.ShapeDtypeStruct((NC, ACC_R, D), jnp.float32),
        mesh=plsc.VectorSubcoreMesh(core_axis_name="c", subcore_axis_name="s",
                                    num_cores=NC, num_subcores=NS),
        compiler_params=pltpu.CompilerParams(needs_layout_passes=False),
        scratch_types=[
            pltpu.VMEM((DEPTH, CCH), jnp.int32),
            pltpu.VMEM((DEPTH, CCH), jnp.int32),
            pltpu.VMEM((DEPTH, CCH), jnp.float32),
            pltpu.VMEM((DEPTH, CCH, D), jnp.float32),
            pltpu.VMEM((2 * VPR, 16), jnp.float32),
            pltpu.VMEM((128,), jnp.int32),
            pltpu.VMEM_SHARED((ACC_R, D), jnp.float32),
        ] + [pltpu.SemaphoreType.DMA] * (3 * DEPTH),
    )


def _edge_aggregate(x, csrc, cdst, cea, cnt, wb):
    return _make_edge_aggregate()(x, csrc, cdst, cea, cnt, wb)


def _agg_from_partials(p_ref):
    return jnp.concatenate(
        [p_ref[0, :HALF], p_ref[1, :N_NODES - HALF]], axis=0)


def _node_body(eps_ref, x_ref, p_ref, W1_ref, b1_ref, g1_ref, be1_ref,
               W2_ref, b2_ref, go_ref, bo_ref, out_ref):
    a = 1.0 + eps_ref[0]
    u = a * x_ref[...] + _agg_from_partials(p_ref)
    h = jnp.dot(u, W1_ref[...], preferred_element_type=jnp.float32) + b1_ref[...]
    m = jnp.mean(h, axis=0, keepdims=True)
    v = jnp.mean((h - m) ** 2, axis=0, keepdims=True)
    h = jnp.maximum(g1_ref[...] * (h - m) * lax.rsqrt(v + 1e-5) + be1_ref[...],
                    0.0)
    h2 = jnp.dot(h, W2_ref[...], preferred_element_type=jnp.float32) + b2_ref[...]
    m2 = jnp.mean(h2, axis=0, keepdims=True)
    v2 = jnp.mean((h2 - m2) ** 2, axis=0, keepdims=True)
    out_ref[...] = jnp.maximum(
        go_ref[...] * (h2 - m2) * lax.rsqrt(v2 + 1e-5) + bo_ref[...], 0.0)


def _node_phase(eps, x, partials, W1, b1, g1, be1, W2, b2, go, bo):
    return pl.pallas_call(
        _node_body,
        out_shape=jax.ShapeDtypeStruct((N_NODES, D), jnp.float32),
        in_specs=[pl.BlockSpec(memory_space=pltpu.SMEM)] +
                 [pl.BlockSpec()] * 10,
    )(eps, x, partials, W1, b1, g1, be1, W2, b2, go, bo)


def _final_body(eps_ref, batch_ref, x_ref, p_ref, W1_ref, b1_ref, g1_ref,
                be1_ref, W2_ref, b2_ref, go_ref, bo_ref, Wl_ref, bl_ref,
                out_ref):
    a = 1.0 + eps_ref[0]
    u = a * x_ref[...] + _agg_from_partials(p_ref)
    h = jnp.dot(u, W1_ref[...], preferred_element_type=jnp.float32) + b1_ref[...]
    m = jnp.mean(h, axis=0, keepdims=True)
    v = jnp.mean((h - m) ** 2, axis=0, keepdims=True)
    h = jnp.maximum(g1_ref[...] * (h - m) * lax.rsqrt(v + 1e-5) + be1_ref[...],
                    0.0)
    h2 = jnp.dot(h, W2_ref[...], preferred_element_type=jnp.float32) + b2_ref[...]
    m2 = jnp.mean(h2, axis=0, keepdims=True)
    v2 = jnp.mean((h2 - m2) ** 2, axis=0, keepdims=True)
    hf = jnp.maximum(
        go_ref[...] * (h2 - m2) * lax.rsqrt(v2 + 1e-5) + bo_ref[...], 0.0)
    onehot = (lax.broadcasted_iota(jnp.int32, (G_GRAPHS, N_NODES), 0)
              == batch_ref[...]).astype(jnp.float32)
    pooled = jnp.dot(onehot, hf, preferred_element_type=jnp.float32)
    out_ref[...] = (jnp.dot(pooled, Wl_ref[...],
                            preferred_element_type=jnp.float32) + bl_ref[...])


def _final_phase(eps, batch, x, partials, W1, b1, g1, be1, W2, b2, go, bo,
                 Wl, bl):
    return pl.pallas_call(
        _final_body,
        out_shape=jax.ShapeDtypeStruct((G_GRAPHS, 10), jnp.float32),
        in_specs=[pl.BlockSpec(memory_space=pltpu.SMEM)] +
                 [pl.BlockSpec()] * 13,
    )(eps, batch, x, partials, W1, b1, g1, be1, W2, b2, go, bo, Wl, bl)


def kernel(x, edge_index, batch, edge_attr,
           We0, be0, eps0, m0W1, m0b1, m0g, m0be, m0W2, m0b2, g0, bb0,
           We1, be1, eps1, m1W1, m1b1, m1g, m1be, m1W2, m1b2, g1, bb1,
           Wl, bl):
    src = edge_index[0].astype(jnp.int32).reshape(NS, NRAW, RAWC)
    dst = edge_index[1].astype(jnp.int32).reshape(NS, NRAW, RAWC)
    ea = edge_attr.reshape(NS, NRAW, RAWC)

    csrc, cdst, cea, cnt = _make_partition()(src, dst, ea)

    def _wb(We, be):
        return jnp.concatenate([We.reshape(VPR, 16), be.reshape(VPR, 16)], 0)

    p0 = _edge_aggregate(x, csrc, cdst, cea, cnt, _wb(We0, be0))
    h = _node_phase(eps0.reshape(1), x, p0,
                    m0W1, m0b1.reshape(1, D), m0g.reshape(1, D),
                    m0be.reshape(1, D), m0W2, m0b2.reshape(1, D),
                    g0.reshape(1, D), bb0.reshape(1, D))
    p1 = _edge_aggregate(h, csrc, cdst, cea, cnt, _wb(We1, be1))
    out = _final_phase(eps1.reshape(1),
                       batch.astype(jnp.int32).reshape(1, N_NODES),
                       h, p1,
                       m1W1, m1b1.reshape(1, D), m1g.reshape(1, D),
                       m1be.reshape(1, D), m1W2, m1b2.reshape(1, D),
                       g1.reshape(1, D), bb1.reshape(1, D), Wl,
                       bl.reshape(1, 10))
    return out
